# baseline (speedup 1.0000x reference)
import functools

import jax
import jax.numpy as jnp
from jax import lax
from jax.experimental import pallas as pl
from jax.experimental.pallas import tpu as pltpu
from jax.experimental.pallas import tpu_sc as plsc

_EMB = 128
_WIN = 64
_LANES = 16
_CH = 128
_NB = 4


def _run(flat_idx, token_table, pos_table):
    N = flat_idx.shape[0]
    V, D = token_table.shape

    info = plsc.get_sparse_core_info()
    NC, NS = info.num_cores, info.num_subcores
    NW = NC * NS
    n_per_w = N // NW
    n_ch = n_per_w // _CH
    rounds = n_ch // _NB

    mesh = plsc.VectorSubcoreMesh(core_axis_name="c", subcore_axis_name="s")

    @functools.partial(
        pl.kernel,
        mesh=mesh,
        out_type=jax.ShapeDtypeStruct((N, D), jnp.float32),
        scratch_types=(
            [pltpu.VMEM((n_per_w,), jnp.int32),
             pltpu.VMEM((_CH,), jnp.int32),
             pltpu.VMEM_SHARED((V, D), jnp.float32),
             pltpu.VMEM_SHARED((_WIN, D), jnp.float32)]
            + [pltpu.VMEM((_CH, D), jnp.float32) for _ in range(_NB)]
            + [pltpu.SemaphoreType.DMA for _ in range(3 * _NB)]
        ),
    )
    def k(idx_hbm, tok_hbm, pos_hbm, out_hbm, idx_all, pidx, tab_sh, pos_sh,
          *bufs_and_sems):
        rows = list(bufs_and_sems[:_NB])
        g1sem = list(bufs_and_sems[_NB:2 * _NB])
        g2sem = list(bufs_and_sems[2 * _NB:3 * _NB])
        osem = list(bufs_and_sems[3 * _NB:])

        sid = lax.axis_index("s")
        wid = sid * NC + lax.axis_index("c")
        base = wid * n_per_w
        v_per_s = V // NS
        pltpu.sync_copy(tok_hbm.at[pl.ds(sid * v_per_s, v_per_s)],
                        tab_sh.at[pl.ds(sid * v_per_s, v_per_s)])

        @pl.when(sid == 0)
        def _():
            pltpu.sync_copy(pos_hbm, pos_sh)

        pltpu.sync_copy(idx_hbm.at[pl.ds(base, n_per_w)], idx_all)
        lanes = lax.iota(jnp.int32, _LANES)
        for j16 in range(_CH // _LANES):
            pidx[pl.ds(j16 * _LANES, _LANES)] = lax.rem(
                lanes + (j16 * _LANES), _WIN)
        plsc.subcore_barrier()

        def tok_gather(lci, b):
            src = tab_sh.at[idx_all.at[pl.ds(lci * _CH, _CH)]]
            return pltpu.make_async_copy(src, rows[b], g1sem[b])

        def pos_gather_start(b):
            pltpu.async_copy(pos_sh.at[pidx], rows[b], g2sem[b], add=True)

        def pos_gather_wait(b):
            pltpu.make_async_copy(pos_sh.at[pidx], rows[b], g2sem[b]).wait()

        def out_copy(lci, b):
            return pltpu.make_async_copy(
                rows[b], out_hbm.at[pl.ds(base + lci * _CH, _CH)], osem[b])

        tok_gather(0, 0).start()
        tok_gather(1, 1).start()
        tok_gather(0, 0).wait()
        pos_gather_start(0)

        def round_body(i, _):
            for b in range(_NB):
                lci = i * _NB + b
                pos_gather_wait(b)
                out_copy(lci, b).start()

                bm = (b + 1) % _NB
                cond1 = True if b < _NB - 1 else (i < rounds - 1)
                if cond1 is True:
                    tok_gather(lci + 1, bm).wait()
                    pos_gather_start(bm)
                else:
                    @pl.when(cond1)
                    def _():
                        tok_gather(lci + 1, bm).wait()
                        pos_gather_start(bm)

                bn = (b + 2) % _NB
                if b < 2:
                    @pl.when(i >= 1)
                    def _():
                        out_copy(lci + 2 - _NB, bn).wait()
                        tok_gather(lci + 2, bn).start()

                    @pl.when(i == 0)
                    def _():
                        tok_gather(lci + 2, bn).start()
                else:
                    @pl.when(i < rounds - 1)
                    def _():
                        out_copy(lci + 2 - _NB, bn).wait()
                        tok_gather(lci + 2, bn).start()

            return 0

        lax.fori_loop(0, rounds, round_body, 0)
        for lci in range(n_ch - _NB, n_ch):
            out_copy(lci, lci % _NB).wait()

    return k(flat_idx, token_table, pos_table)


def kernel(input, token_table, pos_table):
    B, W = input.shape
    D = token_table.shape[1]
    flat_idx = input.reshape(B * W).astype(jnp.int32)
    out = _run(flat_idx, token_table, pos_table)
    return out.reshape(B, W, D)
